# baseline (speedup 1.0000x reference)
import jax
import jax.numpy as jnp
from jax.experimental import pallas as pl

B, NQ, NK = 1, 2048, 1024
D_MSG, D_FF, D_OUT = 16, 32, 16
TQ, TK = 256, 256
G = 8


def _pair_body(q_ref, kw_ref, wbd_ref, b2_ref, o_ref):
    h = jax.lax.dot_general(
        q_ref[...], kw_ref[...], (((1,), (0,)), ((), ())),
        preferred_element_type=jnp.float32)
    s = h * (0.5 * jnp.tanh(h * 0.5) + 0.5)
    sb = s.astype(jnp.bfloat16)
    bias = b2_ref[0:1, :]
    for g in range(TK // G):
        blk = jax.lax.dot_general(
            sb[:, g * (G * D_FF):(g + 1) * (G * D_FF)], wbd_ref[...],
            (((1,), (0,)), ((), ())), preferred_element_type=jnp.float32)
        o_ref[:, g * (G * D_OUT):(g + 1) * (G * D_OUT)] = blk + bias


def kernel(q_equi, q_inv, k_equi, k_inv, Wq, bq, Wk, bk, W1, b1, W2, b2):
    f32 = jnp.float32
    qf = q_equi.reshape(NQ, 24)
    kf = k_equi.reshape(NK, 24)
    qm = q_inv.reshape(NQ, -1) @ Wq + bq
    km = k_inv.reshape(NK, -1) @ Wk + bk
    W1q, W1k, W1d = W1[:16], W1[16:32], W1[32:40]
    bkrow = (km @ W1k + b1).reshape(1, NK * D_FF)

    w1e = jnp.tile(W1d, (3, 1))
    kew_dot = (kf.T[:, :, None] * w1e[:, None, :]).reshape(24, NK * D_FF)
    kew_q = jnp.tile(W1q, (1, NK))
    kew = jnp.concatenate([kew_dot, kew_q, bkrow,
                           jnp.zeros((64 - 41, NK * D_FF), f32)], axis=0)

    qext = jnp.concatenate(
        [qf, qm, jnp.ones((NQ, 1), f32), jnp.zeros((NQ, 64 - 41), f32)],
        axis=1)

    wbd = jnp.zeros((G * D_FF, G * D_OUT), f32)
    for i in range(G):
        wbd = wbd.at[i * D_FF:(i + 1) * D_FF,
                     i * D_OUT:(i + 1) * D_OUT].set(W2)
    b2t = jnp.broadcast_to(jnp.tile(b2, (G,)), (8, G * D_OUT))

    qext = qext.astype(jnp.bfloat16)
    kew = kew.astype(jnp.bfloat16)
    wbd = wbd.astype(jnp.bfloat16)

    out_flat = pl.pallas_call(
        _pair_body,
        grid=(NQ // TQ, NK // TK),
        in_specs=[
            pl.BlockSpec((TQ, 64), lambda iq, ik: (iq, 0)),
            pl.BlockSpec((64, TK * D_FF), lambda iq, ik: (0, ik)),
            pl.BlockSpec((G * D_FF, G * D_OUT), lambda iq, ik: (0, 0)),
            pl.BlockSpec((8, G * D_OUT), lambda iq, ik: (0, 0)),
        ],
        out_specs=pl.BlockSpec((TQ, TK * D_OUT), lambda iq, ik: (iq, ik)),
        out_shape=jax.ShapeDtypeStruct((NQ, NK * D_OUT), f32),
    )(qext, kew, wbd, b2t)

    return out_flat.reshape(B, NQ, NK, D_OUT)


# speedup vs baseline: 2.1855x; 2.1855x over previous
import jax
import jax.numpy as jnp
from jax.experimental import pallas as pl

B, NQ, NK = 1, 2048, 1024
D_MSG, D_FF, D_OUT = 16, 32, 16
TQ = 64


def _pair_body(qw_ref, kx_ref, wbd_ref, b2_ref, o_ref):
    h = jax.lax.dot_general(
        qw_ref[...], kx_ref[...], (((1,), (0,)), ((), ())),
        preferred_element_type=jnp.float32)
    s = h * (0.5 * jnp.tanh(h * 0.5) + 0.5)
    sb = s.astype(jnp.bfloat16)
    bias = b2_ref[:, 0:1]
    for g in range(TQ // 8):
        r = jax.lax.dot_general(
            wbd_ref[...], sb[g * 256:(g + 1) * 256, :],
            (((1,), (0,)), ((), ())), preferred_element_type=jnp.float32)
        o_ref[g * 8:(g + 1) * 8, :, :] = (r + bias).reshape(8, D_OUT, NK)


def kernel(q_equi, q_inv, k_equi, k_inv, Wq, bq, Wk, bk, W1, b1, W2, b2):
    f32 = jnp.float32
    qf = q_equi.reshape(NQ, 24)
    kf_t = k_equi.reshape(NK, 24).T
    km_t = (k_inv.reshape(NK, -1) @ Wk + bk).T
    kx = jnp.concatenate(
        [kf_t, km_t, jnp.ones((1, NK), f32), jnp.zeros((23, NK), f32)],
        axis=0)

    W1q, W1k, W1d = W1[:16], W1[16:32], W1[32:40]
    aq = (q_inv.reshape(NQ, -1) @ Wq + bq) @ W1q + b1
    w1d_t = jnp.tile(W1d, (3, 1)).T
    qw3 = jnp.concatenate([
        qf[:, None, :] * w1d_t[None, :, :],
        jnp.broadcast_to(W1k.T[None, :, :], (NQ, D_FF, 16)),
        aq[:, :, None],
        jnp.zeros((NQ, D_FF, 23), f32)], axis=-1)
    qw = qw3.astype(jnp.bfloat16).reshape(NQ * D_FF, 64)

    wbd = jnp.kron(jnp.eye(8, dtype=f32), W2.T).astype(jnp.bfloat16)
    b2c = jnp.broadcast_to(jnp.tile(b2, (8,))[:, None], (128, 128))

    out_t = pl.pallas_call(
        _pair_body,
        grid=(NQ // TQ,),
        in_specs=[
            pl.BlockSpec((TQ * D_FF, 64), lambda i: (i, 0)),
            pl.BlockSpec((64, NK), lambda i: (0, 0)),
            pl.BlockSpec((128, 256), lambda i: (0, 0)),
            pl.BlockSpec((128, 128), lambda i: (0, 0)),
        ],
        out_specs=pl.BlockSpec((TQ, D_OUT, NK), lambda i: (i, 0, 0)),
        out_shape=jax.ShapeDtypeStruct((NQ, D_OUT, NK), f32),
    )(qw, kx.astype(jnp.bfloat16), wbd, b2c)

    return out_t.reshape(B, NQ, D_OUT, NK).transpose(0, 1, 3, 2)


# speedup vs baseline: 3.6512x; 1.6706x over previous
import jax
import jax.numpy as jnp
from jax.experimental import pallas as pl

B, NQ, NK = 1, 2048, 1024
D_MSG, D_FF, D_OUT = 16, 32, 16
TQ = 64


def _pair_body(a_ref, bt_ref, kx_ref, wbd_ref, b2_ref, o_ref):
    qw = (a_ref[...][:, None, :] * bt_ref[...][None, :, :]).astype(
        jnp.bfloat16).reshape(TQ * D_FF, 64)
    h = jax.lax.dot_general(
        qw, kx_ref[...], (((1,), (0,)), ((), ())),
        preferred_element_type=jnp.float32)
    s = h * (0.5 * jnp.tanh(h * 0.5) + 0.5)
    sb = s.astype(jnp.bfloat16)
    bias = b2_ref[:, 0:1]
    for g in range(TQ // 8):
        r = jax.lax.dot_general(
            wbd_ref[...], sb[g * 256:(g + 1) * 256, :],
            (((1,), (0,)), ((), ())), preferred_element_type=jnp.float32)
        o_ref[g * 8:(g + 1) * 8, :, :] = (r + bias).reshape(8, D_OUT, NK)


def kernel(q_equi, q_inv, k_equi, k_inv, Wq, bq, Wk, bk, W1, b1, W2, b2):
    f32 = jnp.float32
    qf = q_equi.reshape(NQ, 24)
    qm = q_inv.reshape(NQ, -1) @ Wq + bq
    a = jnp.concatenate(
        [qf, jnp.ones((NQ, D_MSG), f32), qm, jnp.ones((NQ, 1), f32),
         jnp.zeros((NQ, 7), f32)], axis=1)

    W1q, W1k, W1d = W1[:16], W1[16:32], W1[32:40]
    bt = jnp.concatenate(
        [jnp.tile(W1d, (3, 1)).T, W1k.T, W1q.T, b1[:, None],
         jnp.zeros((D_FF, 7), f32)], axis=1)

    kf_t = k_equi.reshape(NK, 24).T
    km_t = (k_inv.reshape(NK, -1) @ Wk + bk).T
    kx = jnp.concatenate(
        [kf_t, km_t, jnp.ones((17, NK), f32), jnp.zeros((7, NK), f32)],
        axis=0)

    wbd = jnp.kron(jnp.eye(8, dtype=f32), W2.T).astype(jnp.bfloat16)
    b2c = jnp.broadcast_to(jnp.tile(b2, (8,))[:, None], (128, 128))

    out_t = pl.pallas_call(
        _pair_body,
        grid=(NQ // TQ,),
        in_specs=[
            pl.BlockSpec((TQ, 64), lambda i: (i, 0)),
            pl.BlockSpec((D_FF, 64), lambda i: (0, 0)),
            pl.BlockSpec((64, NK), lambda i: (0, 0)),
            pl.BlockSpec((128, 256), lambda i: (0, 0)),
            pl.BlockSpec((128, 128), lambda i: (0, 0)),
        ],
        out_specs=pl.BlockSpec((TQ, D_OUT, NK), lambda i: (i, 0, 0)),
        out_shape=jax.ShapeDtypeStruct((NQ, D_OUT, NK), f32),
    )(a, bt, kx.astype(jnp.bfloat16), wbd, b2c)

    return out_t.reshape(B, NQ, D_OUT, NK).transpose(0, 1, 3, 2)


# speedup vs baseline: 3.7537x; 1.0281x over previous
import jax
import jax.numpy as jnp
from jax.experimental import pallas as pl

B, NQ, NK = 1, 2048, 1024
D_MSG, D_FF, D_OUT = 16, 32, 16
TQ = 64


def _pair_body(a_ref, bt_ref, kx_ref, wbd_ref, b2_ref, o_ref):
    qw = (a_ref[...][:, None, :] * bt_ref[...][None, :, :]).astype(
        jnp.bfloat16).reshape(TQ * D_FF, 64)
    h = jax.lax.dot_general(
        qw, kx_ref[...], (((1,), (0,)), ((), ())),
        preferred_element_type=jnp.float32)
    u = (h * 0.5).astype(jnp.bfloat16)
    t = jnp.tanh(u)
    sb = u * t + u
    bias = b2_ref[:, 0:1]
    for g in range(TQ // 8):
        r = jax.lax.dot_general(
            wbd_ref[...], sb[g * 256:(g + 1) * 256, :],
            (((1,), (0,)), ((), ())), preferred_element_type=jnp.float32)
        o_ref[g * 8:(g + 1) * 8, :, :] = (r + bias).reshape(8, D_OUT, NK)


def kernel(q_equi, q_inv, k_equi, k_inv, Wq, bq, Wk, bk, W1, b1, W2, b2):
    f32 = jnp.float32
    qf = q_equi.reshape(NQ, 24)
    qm = q_inv.reshape(NQ, -1) @ Wq + bq
    a = jnp.concatenate(
        [qf, jnp.ones((NQ, D_MSG), f32), qm, jnp.ones((NQ, 1), f32),
         jnp.zeros((NQ, 7), f32)], axis=1)

    W1q, W1k, W1d = W1[:16], W1[16:32], W1[32:40]
    bt = jnp.concatenate(
        [jnp.tile(W1d, (3, 1)).T, W1k.T, W1q.T, b1[:, None],
         jnp.zeros((D_FF, 7), f32)], axis=1)

    kf_t = k_equi.reshape(NK, 24).T
    km_t = (k_inv.reshape(NK, -1) @ Wk + bk).T
    kx = jnp.concatenate(
        [kf_t, km_t, jnp.ones((17, NK), f32), jnp.zeros((7, NK), f32)],
        axis=0)

    wbd = jnp.kron(jnp.eye(8, dtype=f32), W2.T).astype(jnp.bfloat16)
    b2c = jnp.broadcast_to(jnp.tile(b2, (8,))[:, None], (128, 128))

    out_t = pl.pallas_call(
        _pair_body,
        grid=(NQ // TQ,),
        in_specs=[
            pl.BlockSpec((TQ, 64), lambda i: (i, 0)),
            pl.BlockSpec((D_FF, 64), lambda i: (0, 0)),
            pl.BlockSpec((64, NK), lambda i: (0, 0)),
            pl.BlockSpec((128, 256), lambda i: (0, 0)),
            pl.BlockSpec((128, 128), lambda i: (0, 0)),
        ],
        out_specs=pl.BlockSpec((TQ, D_OUT, NK), lambda i: (i, 0, 0)),
        out_shape=jax.ShapeDtypeStruct((NQ, D_OUT, NK), f32),
    )(a, bt, kx.astype(jnp.bfloat16), wbd, b2c)

    return out_t.reshape(B, NQ, D_OUT, NK).transpose(0, 1, 3, 2)


# speedup vs baseline: 3.8325x; 1.0210x over previous
import jax
import jax.numpy as jnp
from jax.experimental import pallas as pl

B, NQ, NK = 1, 2048, 1024
D_MSG, D_FF, D_OUT = 16, 32, 16
TQ = 128


def _pair_body(a_ref, bt_ref, kx_ref, wbd_ref, b2_ref, o_ref):
    qw = (a_ref[...][:, None, :] * bt_ref[...][None, :, :]).astype(
        jnp.bfloat16).reshape(TQ * D_FF, 64)
    h = jax.lax.dot_general(
        qw, kx_ref[...], (((1,), (0,)), ((), ())),
        preferred_element_type=jnp.float32)
    u = (h * 0.5).astype(jnp.bfloat16)
    t = jnp.tanh(u)
    sb = u * t + u
    bias = b2_ref[:, 0:1]
    for g in range(TQ // 8):
        r = jax.lax.dot_general(
            wbd_ref[...], sb[g * 256:(g + 1) * 256, :],
            (((1,), (0,)), ((), ())), preferred_element_type=jnp.float32)
        o_ref[g * 8:(g + 1) * 8, :, :] = (r + bias).reshape(8, D_OUT, NK)


def kernel(q_equi, q_inv, k_equi, k_inv, Wq, bq, Wk, bk, W1, b1, W2, b2):
    f32 = jnp.float32
    qf = q_equi.reshape(NQ, 24)
    qm = q_inv.reshape(NQ, -1) @ Wq + bq
    a = jnp.concatenate(
        [qf, jnp.ones((NQ, D_MSG), f32), qm, jnp.ones((NQ, 1), f32),
         jnp.zeros((NQ, 7), f32)], axis=1)

    W1q, W1k, W1d = W1[:16], W1[16:32], W1[32:40]
    bt = jnp.concatenate(
        [jnp.tile(W1d, (3, 1)).T, W1k.T, W1q.T, b1[:, None],
         jnp.zeros((D_FF, 7), f32)], axis=1)

    kf_t = k_equi.reshape(NK, 24).T
    km_t = (k_inv.reshape(NK, -1) @ Wk + bk).T
    kx = jnp.concatenate(
        [kf_t, km_t, jnp.ones((17, NK), f32), jnp.zeros((7, NK), f32)],
        axis=0)

    wbd = jnp.kron(jnp.eye(8, dtype=f32), W2.T).astype(jnp.bfloat16)
    b2c = jnp.broadcast_to(jnp.tile(b2, (8,))[:, None], (128, 128))

    out_t = pl.pallas_call(
        _pair_body,
        grid=(NQ // TQ,),
        in_specs=[
            pl.BlockSpec((TQ, 64), lambda i: (i, 0)),
            pl.BlockSpec((D_FF, 64), lambda i: (0, 0)),
            pl.BlockSpec((64, NK), lambda i: (0, 0)),
            pl.BlockSpec((128, 256), lambda i: (0, 0)),
            pl.BlockSpec((128, 128), lambda i: (0, 0)),
        ],
        out_specs=pl.BlockSpec((TQ, D_OUT, NK), lambda i: (i, 0, 0)),
        out_shape=jax.ShapeDtypeStruct((NQ, D_OUT, NK), f32),
    )(a, bt, kx.astype(jnp.bfloat16), wbd, b2c)

    return out_t.reshape(B, NQ, D_OUT, NK).transpose(0, 1, 3, 2)


# speedup vs baseline: 4.6217x; 1.2059x over previous
import jax
import jax.numpy as jnp
from jax.experimental import pallas as pl

B, NQ, NK = 1, 2048, 1024
D_MSG, D_FF, D_OUT = 16, 32, 16
TQ = 128


def _pair_body(a_ref, bt_ref, kx_ref, wbd_ref, b2_ref, o_ref):
    f8 = jnp.float8_e4m3fn
    qwf = (a_ref[...][:, None, :] * bt_ref[...][None, :, :]).reshape(
        TQ * D_FF, 64) * 64.0
    qhi = qwf.astype(f8)
    qlo = (qwf - qhi.astype(jnp.float32)).astype(f8)
    qw = jnp.concatenate([qhi, qlo, qhi], axis=1)
    h = jax.lax.dot_general(
        qw, kx_ref[...], (((1,), (0,)), ((), ())),
        preferred_element_type=jnp.float32)
    u = (h * (0.5 / 512.0)).astype(jnp.bfloat16)
    t = jnp.tanh(u)
    sb = u * t + u
    bias = b2_ref[:, 0:1]
    for g in range(TQ // 8):
        r = jax.lax.dot_general(
            wbd_ref[...], sb[g * 256:(g + 1) * 256, :],
            (((1,), (0,)), ((), ())), preferred_element_type=jnp.float32)
        o_ref[g * 8:(g + 1) * 8, :, :] = (r + bias).reshape(8, D_OUT, NK)


def kernel(q_equi, q_inv, k_equi, k_inv, Wq, bq, Wk, bk, W1, b1, W2, b2):
    f32 = jnp.float32
    qf = q_equi.reshape(NQ, 24)
    qm = q_inv.reshape(NQ, -1) @ Wq + bq
    a = jnp.concatenate(
        [qf, jnp.ones((NQ, D_MSG), f32), qm, jnp.ones((NQ, 1), f32),
         jnp.zeros((NQ, 7), f32)], axis=1)

    W1q, W1k, W1d = W1[:16], W1[16:32], W1[32:40]
    bt = jnp.concatenate(
        [jnp.tile(W1d, (3, 1)).T, W1k.T, W1q.T, b1[:, None],
         jnp.zeros((D_FF, 7), f32)], axis=1)

    kf_t = k_equi.reshape(NK, 24).T
    km_t = (k_inv.reshape(NK, -1) @ Wk + bk).T
    kx = jnp.concatenate(
        [kf_t, km_t, jnp.ones((17, NK), f32), jnp.zeros((7, NK), f32)],
        axis=0)

    kx = kx * 8.0
    kxhi = kx.astype(jnp.float8_e4m3fn)
    kxlo = (kx - kxhi.astype(f32)).astype(jnp.float8_e4m3fn)
    kx8 = jnp.concatenate([kxhi, kxhi, kxlo], axis=0)

    wbd = jnp.kron(jnp.eye(8, dtype=f32), W2.T).astype(jnp.bfloat16)
    b2c = jnp.broadcast_to(jnp.tile(b2, (8,))[:, None], (128, 128))

    out_t = pl.pallas_call(
        _pair_body,
        grid=(NQ // TQ,),
        in_specs=[
            pl.BlockSpec((TQ, 64), lambda i: (i, 0)),
            pl.BlockSpec((D_FF, 64), lambda i: (0, 0)),
            pl.BlockSpec((192, NK), lambda i: (0, 0)),
            pl.BlockSpec((128, 256), lambda i: (0, 0)),
            pl.BlockSpec((128, 128), lambda i: (0, 0)),
        ],
        out_specs=pl.BlockSpec((TQ, D_OUT, NK), lambda i: (i, 0, 0)),
        out_shape=jax.ShapeDtypeStruct((NQ, D_OUT, NK), f32),
    )(a, bt, kx8, wbd, b2c)

    return out_t.reshape(B, NQ, D_OUT, NK).transpose(0, 1, 3, 2)
